# baseline (speedup 1.0000x reference)
import functools
import math

import jax
import jax.numpy as jnp
from jax import lax
from jax.experimental import pallas as pl
from jax.experimental.pallas import tpu as pltpu
from jax.experimental.pallas import tpu_sc as plsc

NC = 2
NS = 16
NW = NC * NS
ECHUNK = 128


def _npad(n):
    return (n // 128 + 1) * 128


def _make_agg(n_nodes, half, chunks):
    npad = _npad(n_nodes)
    rows_per_tile = npad // NS
    mesh = plsc.VectorSubcoreMesh(core_axis_name="c", subcore_axis_name="s")

    @functools.partial(
        pl.kernel,
        out_type=jax.ShapeDtypeStruct((NC, npad, half), jnp.float32),
        mesh=mesh,
        scratch_types=[
            pltpu.VMEM((chunks, ECHUNK), jnp.int32),
            pltpu.VMEM((chunks, ECHUNK), jnp.int32),
            pltpu.VMEM((3, ECHUNK, half), jnp.float32),
            pltpu.VMEM_SHARED((npad, half), jnp.float32),
            pltpu.SemaphoreType.DMA,
            pltpu.SemaphoreType.DMA,
            pltpu.SemaphoreType.DMA,
            pltpu.SemaphoreType.DMA,
            pltpu.SemaphoreType.DMA,
            pltpu.SemaphoreType.DMA,
        ],
        compiler_params=pltpu.CompilerParams(use_tc_tiling_on_sc=False),
    )
    def agg(h2_hbm, src_hbm, dst_hbm, zero_hbm, out_hbm,
            src_v, dst_v, rows_v, acc_sh, g0, g1, g2, s0, s1, s2):
        c = lax.axis_index("c")
        s = lax.axis_index("s")
        gsems = (g0, g1, g2)
        ssems = (s0, s1, s2)

        pltpu.sync_copy(src_hbm.at[c * NS + s], src_v)
        pltpu.sync_copy(dst_hbm.at[s], dst_v)
        for b in range(2):
            pltpu.async_copy(h2_hbm.at[src_v.at[b]], rows_v.at[b], gsems[b])
        row0 = s * rows_per_tile
        pltpu.sync_copy(zero_hbm.at[pl.ds(row0, rows_per_tile)],
                        acc_sh.at[pl.ds(row0, rows_per_tile)])
        plsc.subcore_barrier()

        def gwait(j, b):
            pltpu.make_async_copy(h2_hbm.at[src_v.at[j]],
                                  rows_v.at[b], gsems[b]).wait()

        def swait(j, b):
            pltpu.make_async_copy(rows_v.at[b],
                                  acc_sh.at[dst_v.at[j]], ssems[b]).wait()

        def triple(q, carry):
            for b in range(3):
                j = 3 * q + b
                gwait(j, b)
                pltpu.async_copy(rows_v.at[b], acc_sh.at[dst_v.at[j]],
                                 ssems[b], add=True)
                bn = (b + 2) % 3
                if b == 0:
                    @pl.when(q > 0)
                    def _():
                        swait(j - 1, bn)
                else:
                    swait(j - 1, bn)
                nxt = j + 2

                @pl.when(nxt < chunks)
                def _():
                    pltpu.async_copy(h2_hbm.at[src_v.at[nxt]],
                                     rows_v.at[bn], gsems[bn])
            return carry

        lax.fori_loop(0, chunks // 3, triple, 0)
        swait(chunks - 1, 2)
        plsc.subcore_barrier()
        pltpu.sync_copy(acc_sh.at[pl.ds(row0, rows_per_tile)],
                        out_hbm.at[c, pl.ds(row0, rows_per_tile)])

    return agg


def _silu(v):
    return v * jax.nn.sigmoid(v)


def kernel(x, noise_graph_X_t, edge_index, t,
           W_t0, b_t0, W_t1, b_t1,
           W_d0, b_d0, W_d1, b_d1,
           W_u0, b_u0, W_u1, b_u1):
    n = x.shape[0]
    d = x.shape[1]
    e = edge_index.shape[1]
    npad = _npad(n)

    chunks = 3 * (-(-e // (NS * ECHUNK * 3)))
    ep = NS * ECHUNK * chunks
    pad = ep - e
    src = jnp.concatenate([edge_index[0], jnp.zeros((pad,), jnp.int32)])
    dst = jnp.concatenate([edge_index[1], jnp.full((pad,), n, jnp.int32)])
    src2 = (2 * src)[None, :] + jnp.arange(2, dtype=jnp.int32)[:, None]
    src2 = src2.reshape(2 * NS, chunks, ECHUNK)
    dst16 = dst.reshape(NS, chunks, ECHUNK)

    zeros64 = jnp.zeros((npad, d // 2), jnp.float32)
    zeros32 = jnp.zeros((npad, d // 4), jnp.float32)
    zeros8 = jnp.zeros((npad, 8), jnp.float32)
    ones8 = jnp.ones((2 * n, 8), jnp.float32)

    agg128 = _make_agg(n, d // 2, chunks)
    agg64 = _make_agg(n, d // 4, chunks)
    agg8 = _make_agg(n, 8, chunks)

    def agg_full(p, aggk):
        f = p.shape[1]
        u = aggk(p.reshape(2 * n, f // 2), src2, dst16,
                 zeros64 if f == d else zeros32)
        return jnp.concatenate([u[0, :n], u[1, :n]], axis=1)

    degp = agg8(ones8, src2, dst16, zeros8)
    deg = degp[0, :n, 0] + 1.0
    dis = lax.rsqrt(deg)[:, None]

    half = d // 2
    freq = jnp.exp(jnp.arange(half, dtype=jnp.float32)
                   * (-math.log(10000.0) / (half - 1)))
    ang = t.astype(jnp.float32)[:, None] * freq[None, :]
    emb = jnp.concatenate([jnp.sin(ang), jnp.cos(ang)], axis=1)
    emb = _silu(emb @ W_t0 + b_t0) @ W_t1 + b_t1
    x_t = noise_graph_X_t + emb

    p1 = dis * (x_t @ W_d0)
    h1 = _silu(dis * (agg_full(p1, agg128) + p1) + b_d0)
    p2 = dis * (h1 @ W_d1)
    h2 = _silu(dis * (agg_full(p2, agg64) + p2) + b_d1)
    p3 = dis * h2
    h3 = _silu(dis * ((agg_full(p3, agg64) + p3) @ W_u0) + b_u0)
    p4 = dis * (h3 @ W_u1[:d] + h1 @ W_u1[d:])
    h4 = dis * (agg_full(p4, agg128) + p4) + b_u1
    return _silu(h4)


# speedup vs baseline: 1.1614x; 1.1614x over previous
import functools
import math

import jax
import jax.numpy as jnp
from jax import lax
from jax.experimental import pallas as pl
from jax.experimental.pallas import tpu as pltpu
from jax.experimental.pallas import tpu_sc as plsc

NC = 2
NS = 16
NW = NC * NS
ECHUNK = 128


def _npad(n):
    return (n // 128 + 1) * 128


def _make_agg(n_nodes, half, chunks):
    npad = _npad(n_nodes)
    rows_per_tile = npad // NS
    mesh = plsc.VectorSubcoreMesh(core_axis_name="c", subcore_axis_name="s")

    @functools.partial(
        pl.kernel,
        out_type=jax.ShapeDtypeStruct((NC, npad, half), jnp.float32),
        mesh=mesh,
        scratch_types=[
            pltpu.VMEM((chunks, ECHUNK), jnp.int32),
            pltpu.VMEM((chunks, ECHUNK), jnp.int32),
            pltpu.VMEM((2, ECHUNK, half), jnp.float32),
            pltpu.VMEM_SHARED((npad, half), jnp.float32),
            pltpu.SemaphoreType.DMA,
            pltpu.SemaphoreType.DMA,
        ],
        compiler_params=pltpu.CompilerParams(use_tc_tiling_on_sc=False),
    )
    def agg(h2_hbm, src_hbm, dst_hbm, zero_hbm, out_hbm,
            src_v, dst_v, rows_v, acc_sh, sem0, sem1):
        c = lax.axis_index("c")
        s = lax.axis_index("s")
        sems = (sem0, sem1)

        pltpu.sync_copy(src_hbm.at[c * NS + s], src_v)
        pltpu.sync_copy(dst_hbm.at[s], dst_v)
        for b in range(2):
            pltpu.async_copy(h2_hbm.at[src_v.at[b]], rows_v.at[b], sems[b])
        row0 = s * rows_per_tile
        pltpu.sync_copy(zero_hbm.at[pl.ds(row0, rows_per_tile)],
                        acc_sh.at[pl.ds(row0, rows_per_tile)])
        plsc.subcore_barrier()

        def pair(p, carry):
            for b in range(2):
                j = 2 * p + b
                pltpu.make_async_copy(h2_hbm.at[src_v.at[j]],
                                      rows_v.at[b], sems[b]).wait()
                pltpu.sync_copy(rows_v.at[b], acc_sh.at[dst_v.at[j]],
                                add=True)
                nxt = j + 2

                @pl.when(nxt < chunks)
                def _():
                    pltpu.async_copy(h2_hbm.at[src_v.at[nxt]],
                                     rows_v.at[b], sems[b])
            return carry

        lax.fori_loop(0, chunks // 2, pair, 0)
        plsc.subcore_barrier()
        pltpu.sync_copy(acc_sh.at[pl.ds(row0, rows_per_tile)],
                        out_hbm.at[c, pl.ds(row0, rows_per_tile)])

    return agg


def _silu(v):
    return v * jax.nn.sigmoid(v)


def kernel(x, noise_graph_X_t, edge_index, t,
           W_t0, b_t0, W_t1, b_t1,
           W_d0, b_d0, W_d1, b_d1,
           W_u0, b_u0, W_u1, b_u1):
    n = x.shape[0]
    d = x.shape[1]
    e = edge_index.shape[1]
    npad = _npad(n)

    chunks = 2 * (-(-e // (NS * ECHUNK * 2)))
    ep = NS * ECHUNK * chunks
    pad = ep - e
    src = jnp.concatenate([edge_index[0], jnp.zeros((pad,), jnp.int32)])
    dst = jnp.concatenate([edge_index[1], jnp.full((pad,), n, jnp.int32)])
    src2 = (2 * src)[None, :] + jnp.arange(2, dtype=jnp.int32)[:, None]
    src2 = src2.reshape(2 * NS, chunks, ECHUNK)
    dst16 = dst.reshape(NS, chunks, ECHUNK)

    zeros64 = jnp.zeros((npad, d // 2), jnp.float32)
    zeros32 = jnp.zeros((npad, d // 4), jnp.float32)
    zeros8 = jnp.zeros((npad, 8), jnp.float32)
    ones8 = jnp.ones((2 * n, 8), jnp.float32)

    agg128 = _make_agg(n, d // 2, chunks)
    agg64 = _make_agg(n, d // 4, chunks)
    agg8 = _make_agg(n, 8, chunks)

    def agg_full(p, aggk):
        f = p.shape[1]
        u = aggk(p.reshape(2 * n, f // 2), src2, dst16,
                 zeros64 if f == d else zeros32)
        return jnp.concatenate([u[0, :n], u[1, :n]], axis=1)

    degp = agg8(ones8, src2, dst16, zeros8)
    deg = degp[0, :n, 0] + 1.0
    dis = lax.rsqrt(deg)[:, None]

    half = d // 2
    freq = jnp.exp(jnp.arange(half, dtype=jnp.float32)
                   * (-math.log(10000.0) / (half - 1)))
    ang = t.astype(jnp.float32)[:, None] * freq[None, :]
    emb = jnp.concatenate([jnp.sin(ang), jnp.cos(ang)], axis=1)
    emb = _silu(emb @ W_t0 + b_t0) @ W_t1 + b_t1
    x_t = noise_graph_X_t + emb

    p1 = dis * (x_t @ W_d0)
    h1 = _silu(dis * (agg_full(p1, agg128) + p1) + b_d0)
    p2 = dis * (h1 @ W_d1)
    h2 = _silu(dis * (agg_full(p2, agg64) + p2) + b_d1)
    p3 = dis * h2
    h3 = _silu(dis * ((agg_full(p3, agg64) + p3) @ W_u0) + b_u0)
    p4 = dis * (h3 @ W_u1[:d] + h1 @ W_u1[d:])
    h4 = dis * (agg_full(p4, agg128) + p4) + b_u1
    return _silu(h4)
